# baseline (speedup 1.0000x reference)
import functools

import jax
import jax.numpy as jnp
from jax import lax
from jax.experimental import pallas as pl
from jax.experimental.pallas import tpu as pltpu
from jax.experimental.pallas import tpu_sc as plsc

E = 8
TOPK = 2
D = 768
DI = D // 2
F = 4 * D
T = 2048
BM = 512
NB = 16
P = NB * BM
NC, NS = 2, 16
NW = NC * NS
RPW = P // NW
RC = 128
NRC = RPW // RC
TPW = T // NW

_SQRT1_2 = 0.7071067811865476


def _router_kernel(x_ref, rwt_ref, logits_ref, eidx_ref, g_ref):
    x = x_ref[...]
    logits = jnp.dot(x, rwt_ref[...], preferred_element_type=jnp.float32)
    logits_ref[...] = logits
    m = jnp.max(logits, axis=1, keepdims=True)
    ex = jnp.exp(logits - m)
    probs = ex / jnp.sum(ex, axis=1, keepdims=True)
    eix = lax.broadcasted_iota(jnp.int32, probs.shape, 1)
    m1 = jnp.max(probs, axis=1, keepdims=True)
    i1 = jnp.min(jnp.where(probs == m1, eix, E), axis=1, keepdims=True)
    masked = jnp.where(eix == i1, -jnp.inf, probs)
    m2 = jnp.max(masked, axis=1, keepdims=True)
    i2 = jnp.min(jnp.where(masked == m2, eix, E), axis=1, keepdims=True)
    s = m1 + m2
    eidx_ref[...] = jnp.concatenate([i1, i2], axis=1)
    g_ref[...] = jnp.concatenate([m1 / s, m2 / s], axis=1)


def _ffn_kernel(be_ref, nvb_ref, x_ref, w1_ref, w2_ref, g_ref, y_ref):
    del be_ref
    b = pl.program_id(0)

    @pl.when(b < nvb_ref[0])
    def _():
        x = x_ref[...]
        w1 = w1_ref[...].astype(jnp.bfloat16)
        h = jnp.dot(x, w1, preferred_element_type=jnp.float32)
        h = h * 0.5 * (1.0 + lax.erf(h * _SQRT1_2))
        g = g_ref[0, 0, :]
        h = (h * g[:, None]).astype(jnp.bfloat16)
        w2 = w2_ref[...].astype(jnp.bfloat16)
        y_ref[...] = jnp.dot(h, w2, preferred_element_type=jnp.float32)


@functools.lru_cache(maxsize=None)
def _sc_gather_kernel():
    mesh = plsc.VectorSubcoreMesh(core_axis_name="c", subcore_axis_name="s")

    @functools.partial(
        pl.kernel, mesh=mesh,
        out_type=jax.ShapeDtypeStruct((P, DI), jnp.int32),
        scratch_types=[
            pltpu.VMEM((RC,), jnp.int32),
            pltpu.VMEM((RC,), jnp.int32),
            pltpu.VMEM((RPW, DI), jnp.int32),
            pltpu.SemaphoreType.DMA,
        ],
    )
    def k(x_hbm, idx_hbm, out_hbm, idx0_v, idx1_v, rows_v, sem):
        wid = lax.axis_index("s") * NC + lax.axis_index("c")
        base = wid * RPW
        copies = []
        for c, idx_v in enumerate((idx0_v, idx1_v)):
            pltpu.sync_copy(idx_hbm.at[pl.ds(base + c * RC, RC)], idx_v)
            copies.append(pltpu.async_copy(
                x_hbm.at[idx_v], rows_v.at[pl.ds(c * RC, RC)], sem))
        for cp in copies:
            cp.wait()
        pltpu.sync_copy(rows_v, out_hbm.at[pl.ds(base, RPW)])

    return k


@functools.lru_cache(maxsize=None)
def _sc_combine_kernel():
    mesh = plsc.VectorSubcoreMesh(core_axis_name="c", subcore_axis_name="s")

    @functools.partial(
        pl.kernel, mesh=mesh,
        out_type=jax.ShapeDtypeStruct((T, D), jnp.float32),
        scratch_types=[
            pltpu.VMEM((TPW,), jnp.int32),
            pltpu.VMEM((TPW,), jnp.int32),
            pltpu.VMEM((TPW, D), jnp.float32),
            pltpu.VMEM((TPW, D), jnp.float32),
            pltpu.SemaphoreType.DMA,
        ],
    )
    def k(y_hbm, d0_hbm, d1_hbm, out_hbm, i0_v, i1_v, b0_v, b1_v, sem):
        wid = lax.axis_index("s") * NC + lax.axis_index("c")
        base = wid * TPW
        pltpu.sync_copy(d0_hbm.at[pl.ds(base, TPW)], i0_v)
        pltpu.sync_copy(d1_hbm.at[pl.ds(base, TPW)], i1_v)
        c0 = pltpu.async_copy(y_hbm.at[i0_v], b0_v, sem)
        c1 = pltpu.async_copy(y_hbm.at[i1_v], b1_v, sem)
        c0.wait()
        c1.wait()

        def _row(r, carry):
            for c in range(D // 16):
                sl = pl.ds(c * 16, 16)
                b0_v[r, sl] = b0_v[r, sl] + b1_v[r, sl]
            return carry

        lax.fori_loop(0, TPW, _row, 0)
        pltpu.sync_copy(b0_v, out_hbm.at[pl.ds(base, TPW)])

    return k


def _gather_rows(xi, src_tok):
    return _sc_gather_kernel()(xi, src_tok)


def _combine_rows(y, d0, d1):
    return _sc_combine_kernel()(y, d0, d1)


def kernel(x, router_w, w1, w2):
    B, S, _ = x.shape
    xf = x.reshape(T, D)

    logits, eidx, gates = pl.pallas_call(
        _router_kernel,
        out_shape=(
            jax.ShapeDtypeStruct((T, E), jnp.float32),
            jax.ShapeDtypeStruct((T, TOPK), jnp.int32),
            jax.ShapeDtypeStruct((T, TOPK), jnp.float32),
        ),
    )(xf, router_w.T)

    eflat = eidx.reshape(-1)
    gflat = gates.reshape(-1)
    onehot = (eflat[:, None] == jnp.arange(E, dtype=jnp.int32)).astype(jnp.int32)
    csum = jnp.cumsum(onehot, axis=0)
    rank = jnp.take_along_axis(csum, eflat[:, None], axis=1)[:, 0] - 1
    counts = csum[-1]
    pcounts = ((counts + BM - 1) // BM) * BM
    cum_p = jnp.cumsum(pcounts)
    pstart = cum_p - pcounts
    pad_pos = pstart[eflat] + rank
    dst = pad_pos.reshape(T, TOPK)
    pair_tok = jnp.arange(T * TOPK, dtype=jnp.int32) // TOPK
    src_tok = jnp.zeros((P,), jnp.int32).at[pad_pos].set(
        pair_tok, mode="drop", unique_indices=True)
    gate_sorted = jnp.zeros((P,), jnp.float32).at[pad_pos].set(
        gflat, mode="drop", unique_indices=True)
    be_raw = jnp.searchsorted(cum_p, jnp.arange(NB, dtype=jnp.int32) * BM,
                              side="right").astype(jnp.int32)
    last_e = jnp.searchsorted(cum_p, cum_p[-1] - 1,
                              side="right").astype(jnp.int32)
    block_expert = jnp.where(be_raw >= E, last_e, be_raw)
    nvb = (cum_p[-1:] // BM).astype(jnp.int32)

    xi = lax.bitcast_convert_type(
        xf.astype(jnp.bfloat16).reshape(T, DI, 2), jnp.int32)
    xs_i = _gather_rows(xi, src_tok)
    x_sorted = lax.bitcast_convert_type(xs_i, jnp.bfloat16).reshape(P, D)

    grid_spec = pltpu.PrefetchScalarGridSpec(
        num_scalar_prefetch=2,
        grid=(NB,),
        in_specs=[
            pl.BlockSpec((BM, D), lambda b, be, nv: (b, 0)),
            pl.BlockSpec((D, F), lambda b, be, nv: (0, be[b])),
            pl.BlockSpec((F, D), lambda b, be, nv: (be[b], 0)),
            pl.BlockSpec((1, 1, BM), lambda b, be, nv: (b, 0, 0)),
        ],
        out_specs=pl.BlockSpec((BM, D), lambda b, be, nv: (b, 0)),
    )
    y = pl.pallas_call(
        _ffn_kernel,
        grid_spec=grid_spec,
        out_shape=jax.ShapeDtypeStruct((P, D), jnp.float32),
    )(block_expert, nvb, x_sorted, w1, w2, gate_sorted.reshape(NB, 1, BM))

    out = _combine_rows(y, dst[:, 0], dst[:, 1])
    return out.reshape(B, S, D), logits


# speedup vs baseline: 1.5177x; 1.5177x over previous
import functools

import jax
import jax.numpy as jnp
from jax import lax
from jax.experimental import pallas as pl
from jax.experimental.pallas import tpu as pltpu
from jax.experimental.pallas import tpu_sc as plsc

E = 8
TOPK = 2
D = 768
DI = D // 2
F = 4 * D
T = 2048
BM = 512
NB = 16
P = NB * BM
NC, NS = 2, 16
NW = NC * NS
RPW = P // NW
RC = 128
NRC = RPW // RC
TPW = T // NW

_SQRT1_2 = 0.7071067811865476


def _router_kernel(x_ref, rwt_ref, logits_ref, eidx_ref, g_ref):
    x = x_ref[...]
    logits = jnp.dot(x, rwt_ref[...], preferred_element_type=jnp.float32)
    logits_ref[...] = logits
    m = jnp.max(logits, axis=1, keepdims=True)
    ex = jnp.exp(logits - m)
    probs = ex / jnp.sum(ex, axis=1, keepdims=True)
    eix = lax.broadcasted_iota(jnp.int32, probs.shape, 1)
    m1 = jnp.max(probs, axis=1, keepdims=True)
    i1 = jnp.min(jnp.where(probs == m1, eix, E), axis=1, keepdims=True)
    masked = jnp.where(eix == i1, -jnp.inf, probs)
    m2 = jnp.max(masked, axis=1, keepdims=True)
    i2 = jnp.min(jnp.where(masked == m2, eix, E), axis=1, keepdims=True)
    s = m1 + m2
    eidx_ref[...] = jnp.concatenate([i1, i2], axis=1)
    g_ref[...] = jnp.concatenate([m1 / s, m2 / s], axis=1)


def _ffn_kernel(be_ref, nvb_ref, x_ref, w1_ref, w2_ref, g_ref, y_ref):
    del be_ref
    b = pl.program_id(0)

    @pl.when(b < nvb_ref[0])
    def _():
        x = x_ref[...]
        w1 = w1_ref[...].astype(jnp.bfloat16)
        h = jnp.dot(x, w1, preferred_element_type=jnp.float32)
        h = h * 0.5 * (1.0 + lax.erf(h * _SQRT1_2))
        g = g_ref[0, 0, :]
        h = (h * g[:, None]).astype(jnp.bfloat16)
        w2 = w2_ref[...].astype(jnp.bfloat16)
        y_ref[...] = jnp.dot(h, w2, preferred_element_type=jnp.float32)


@functools.lru_cache(maxsize=None)
def _sc_gather_kernel():
    mesh = plsc.VectorSubcoreMesh(core_axis_name="c", subcore_axis_name="s")

    @functools.partial(
        pl.kernel, mesh=mesh,
        out_type=jax.ShapeDtypeStruct((P, DI), jnp.int32),
        scratch_types=[
            pltpu.VMEM((RC,), jnp.int32),
            pltpu.VMEM((RC,), jnp.int32),
            pltpu.VMEM((RPW, DI), jnp.int32),
            pltpu.SemaphoreType.DMA,
        ],
    )
    def k(x_hbm, idx_hbm, out_hbm, idx0_v, idx1_v, rows_v, sem):
        wid = lax.axis_index("s") * NC + lax.axis_index("c")
        base = wid * RPW
        copies = []
        for c, idx_v in enumerate((idx0_v, idx1_v)):
            pltpu.sync_copy(idx_hbm.at[pl.ds(base + c * RC, RC)], idx_v)
            copies.append(pltpu.async_copy(
                x_hbm.at[idx_v], rows_v.at[pl.ds(c * RC, RC)], sem))
        for cp in copies:
            cp.wait()
        pltpu.sync_copy(rows_v, out_hbm.at[pl.ds(base, RPW)])

    return k


@functools.lru_cache(maxsize=None)
def _sc_combine_kernel():
    mesh = plsc.VectorSubcoreMesh(core_axis_name="c", subcore_axis_name="s")

    @functools.partial(
        pl.kernel, mesh=mesh,
        out_type=jax.ShapeDtypeStruct((T, D), jnp.float32),
        scratch_types=[
            pltpu.VMEM((TPW,), jnp.int32),
            pltpu.VMEM((TPW,), jnp.int32),
            pltpu.VMEM((TPW, D), jnp.float32),
            pltpu.VMEM((TPW, D), jnp.float32),
            pltpu.SemaphoreType.DMA,
        ],
    )
    def k(y_hbm, d0_hbm, d1_hbm, out_hbm, i0_v, i1_v, b0_v, b1_v, sem):
        wid = lax.axis_index("s") * NC + lax.axis_index("c")
        base = wid * TPW
        pltpu.sync_copy(d0_hbm.at[pl.ds(base, TPW)], i0_v)
        pltpu.sync_copy(d1_hbm.at[pl.ds(base, TPW)], i1_v)
        c0 = pltpu.async_copy(y_hbm.at[i0_v], b0_v, sem)
        c1 = pltpu.async_copy(y_hbm.at[i1_v], b1_v, sem)
        c0.wait()
        c1.wait()

        def _row(r, carry):
            for c in range(D // 16):
                sl = pl.ds(c * 16, 16)
                b0_v[r, sl] = b0_v[r, sl] + b1_v[r, sl]
            return carry

        lax.fori_loop(0, TPW, _row, 0)
        pltpu.sync_copy(b0_v, out_hbm.at[pl.ds(base, TPW)])

    return k


def _gather_rows(xi, src_tok):
    return _sc_gather_kernel()(xi, src_tok)


def _combine_rows(y, d0, d1):
    return _sc_combine_kernel()(y, d0, d1)


def kernel(x, router_w, w1, w2):
    B, S, _ = x.shape
    xf = x.reshape(T, D)

    logits, eidx, gates = pl.pallas_call(
        _router_kernel,
        out_shape=(
            jax.ShapeDtypeStruct((T, E), jnp.float32),
            jax.ShapeDtypeStruct((T, TOPK), jnp.int32),
            jax.ShapeDtypeStruct((T, TOPK), jnp.float32),
        ),
    )(xf, router_w.T)

    eflat = eidx.reshape(-1)
    gflat = gates.reshape(-1)
    onehot = (eflat[:, None] == jnp.arange(E, dtype=jnp.int32)).astype(jnp.int32)
    csum = jnp.cumsum(onehot, axis=0)
    rank = jnp.take_along_axis(csum, eflat[:, None], axis=1)[:, 0] - 1
    counts = csum[-1]
    pcounts = ((counts + BM - 1) // BM) * BM
    cum_p = jnp.cumsum(pcounts)
    pstart = cum_p - pcounts
    pad_pos = pstart[eflat] + rank
    dst = pad_pos.reshape(T, TOPK)
    pair_tok = jnp.arange(T * TOPK, dtype=jnp.int32) // TOPK
    src_tok = (jnp.arange(P, dtype=jnp.int32) % T).at[pad_pos].set(
        pair_tok, mode="drop", unique_indices=True)
    gate_sorted = jnp.zeros((P,), jnp.float32).at[pad_pos].set(
        gflat, mode="drop", unique_indices=True)
    be_raw = jnp.searchsorted(cum_p, jnp.arange(NB, dtype=jnp.int32) * BM,
                              side="right").astype(jnp.int32)
    last_e = jnp.searchsorted(cum_p, cum_p[-1] - 1,
                              side="right").astype(jnp.int32)
    block_expert = jnp.where(be_raw >= E, last_e, be_raw)
    nvb = (cum_p[-1:] // BM).astype(jnp.int32)

    xi = lax.bitcast_convert_type(
        xf.astype(jnp.bfloat16).reshape(T, DI, 2), jnp.int32)
    xs_i = _gather_rows(xi, src_tok)
    x_sorted = lax.bitcast_convert_type(xs_i, jnp.bfloat16).reshape(P, D)

    grid_spec = pltpu.PrefetchScalarGridSpec(
        num_scalar_prefetch=2,
        grid=(NB,),
        in_specs=[
            pl.BlockSpec((BM, D), lambda b, be, nv: (b, 0)),
            pl.BlockSpec((D, F), lambda b, be, nv: (0, be[b])),
            pl.BlockSpec((F, D), lambda b, be, nv: (be[b], 0)),
            pl.BlockSpec((1, 1, BM), lambda b, be, nv: (b, 0, 0)),
        ],
        out_specs=pl.BlockSpec((BM, D), lambda b, be, nv: (b, 0)),
    )
    y = pl.pallas_call(
        _ffn_kernel,
        grid_spec=grid_spec,
        out_shape=jax.ShapeDtypeStruct((P, D), jnp.float32),
    )(block_expert, nvb, x_sorted, w1, w2, gate_sorted.reshape(NB, 1, BM))

    out = _combine_rows(y, dst[:, 0], dst[:, 1])
    return out.reshape(B, S, D), logits


# speedup vs baseline: 2.9638x; 1.9528x over previous
import functools

import jax
import jax.numpy as jnp
from jax import lax
from jax.experimental import pallas as pl
from jax.experimental.pallas import tpu as pltpu
from jax.experimental.pallas import tpu_sc as plsc

E = 8
TOPK = 2
D = 768
F = 4 * D
T = 2048
NP = T * TOPK
BM = 512
NB = 16
P = NB * BM
NC, NS = 2, 16
NW = NC * NS
RPW = P // NW
RC = 128
SPW = NP // NS
TPW = T // NW

_SQRT1_2 = 0.7071067811865476


def _router_kernel(x_ref, rwt_ref, logits_ref, eidx_ref, g_ref):
    x = x_ref[...]
    logits = jnp.dot(x, rwt_ref[...], preferred_element_type=jnp.float32)
    logits_ref[...] = logits
    m = jnp.max(logits, axis=1, keepdims=True)
    ex = jnp.exp(logits - m)
    probs = ex / jnp.sum(ex, axis=1, keepdims=True)
    eix = lax.broadcasted_iota(jnp.int32, probs.shape, 1)
    m1 = jnp.max(probs, axis=1, keepdims=True)
    i1 = jnp.min(jnp.where(probs == m1, eix, E), axis=1, keepdims=True)
    masked = jnp.where(eix == i1, -jnp.inf, probs)
    m2 = jnp.max(masked, axis=1, keepdims=True)
    i2 = jnp.min(jnp.where(masked == m2, eix, E), axis=1, keepdims=True)
    s = m1 + m2
    eidx_ref[...] = jnp.concatenate([i1, i2], axis=1)
    g_ref[...] = jnp.concatenate([m1 / s, m2 / s], axis=1)


def _ffn_kernel(be_ref, nvb_ref, x_ref, w1_ref, w2_ref, g_ref, y_ref):
    del be_ref
    b = pl.program_id(0)

    @pl.when(b < nvb_ref[0])
    def _():
        x = x_ref[...].astype(jnp.bfloat16)
        w1 = w1_ref[...].astype(jnp.bfloat16)
        h = jnp.dot(x, w1, preferred_element_type=jnp.float32)
        h = h * 0.5 * (1.0 + lax.erf(h * _SQRT1_2))
        g = g_ref[0, 0, :]
        h = (h * g[:, None]).astype(jnp.bfloat16)
        w2 = w2_ref[...].astype(jnp.bfloat16)
        y_ref[...] = jnp.dot(h, w2, preferred_element_type=jnp.float32)


@functools.lru_cache(maxsize=None)
def _sc_dispatch_kernel():
    mesh = plsc.VectorSubcoreMesh(core_axis_name="c", subcore_axis_name="s")

    @functools.partial(
        pl.kernel, mesh=mesh,
        out_type=(
            jax.ShapeDtypeStruct((P, D), jnp.float32),
            jax.ShapeDtypeStruct((P,), jnp.float32),
        ),
        scratch_types=[
            pltpu.VMEM((SPW // RC, RC), jnp.int32),
            pltpu.VMEM((SPW,), jnp.int32),
            pltpu.VMEM((SPW,), jnp.float32),
            pltpu.VMEM((RPW,), jnp.int32),
            pltpu.VMEM((RPW,), jnp.float32),
            pltpu.VMEM((RC, D), jnp.float32),
            pltpu.VMEM_SHARED((P,), jnp.int32),
            pltpu.VMEM_SHARED((P,), jnp.float32),
            pltpu.SemaphoreType.DMA,
        ],
    )
    def k(x_hbm, pp_hbm, tok1_hbm, g_hbm, zi_hbm, zf_hbm,
          xs_hbm, gs_hbm,
          pp_v, tv_v, gv_v, stok_v, sg_v, rows_v, st_sh, gt_sh, sem):
        cid = lax.axis_index("c")
        sid = lax.axis_index("s")
        wid = sid * NC + cid
        sbase = sid * SPW
        obase = wid * RPW

        zlen = P // NS
        pltpu.sync_copy(zi_hbm.at[pl.ds(sid * zlen, zlen)],
                        st_sh.at[pl.ds(sid * zlen, zlen)])
        pltpu.sync_copy(zf_hbm.at[pl.ds(sid * zlen, zlen)],
                        gt_sh.at[pl.ds(sid * zlen, zlen)])
        pltpu.sync_copy(pp_hbm.at[sid], pp_v)
        pltpu.sync_copy(tok1_hbm.at[pl.ds(sbase, SPW)], tv_v)
        pltpu.sync_copy(g_hbm.at[pl.ds(sbase, SPW)], gv_v)
        plsc.subcore_barrier()
        for c in range(SPW // RC):
            sl = pl.ds(c * RC, RC)
            pltpu.sync_copy(tv_v.at[sl], st_sh.at[pp_v.at[c]], add=True)
            pltpu.sync_copy(gv_v.at[sl], gt_sh.at[pp_v.at[c]], add=True)
        plsc.subcore_barrier()
        pltpu.sync_copy(st_sh.at[pl.ds(obase, RPW)], stok_v)
        pltpu.sync_copy(gt_sh.at[pl.ds(obase, RPW)], sg_v)
        pltpu.sync_copy(sg_v, gs_hbm.at[pl.ds(obase, RPW)])
        for c in range(RPW // 16):
            sl = pl.ds(c * 16, 16)
            v = stok_v[sl]
            fb = lax.iota(jnp.int32, 16) + ((obase + c * 16) & (T - 1))
            stok_v[sl] = jnp.where(v > 0, v - 1, fb & (T - 1))
        for c in range(RPW // RC):
            pltpu.async_copy(
                x_hbm.at[stok_v.at[pl.ds(c * RC, RC)]], rows_v, sem).wait()
            pltpu.sync_copy(rows_v, xs_hbm.at[pl.ds(obase + c * RC, RC)])

    return k


@functools.lru_cache(maxsize=None)
def _sc_combine_kernel():
    mesh = plsc.VectorSubcoreMesh(core_axis_name="c", subcore_axis_name="s")

    @functools.partial(
        pl.kernel, mesh=mesh,
        out_type=jax.ShapeDtypeStruct((T, D), jnp.float32),
        scratch_types=[
            pltpu.VMEM((TPW,), jnp.int32),
            pltpu.VMEM((TPW,), jnp.int32),
            pltpu.VMEM((TPW, D), jnp.float32),
            pltpu.VMEM((TPW, D), jnp.float32),
            pltpu.SemaphoreType.DMA,
        ],
    )
    def k(y_hbm, d0_hbm, d1_hbm, out_hbm, i0_v, i1_v, b0_v, b1_v, sem):
        wid = lax.axis_index("s") * NC + lax.axis_index("c")
        base = wid * TPW
        pltpu.sync_copy(d0_hbm.at[pl.ds(base, TPW)], i0_v)
        pltpu.sync_copy(d1_hbm.at[pl.ds(base, TPW)], i1_v)
        c0 = pltpu.async_copy(y_hbm.at[i0_v], b0_v, sem)
        c1 = pltpu.async_copy(y_hbm.at[i1_v], b1_v, sem)
        c0.wait()
        c1.wait()

        def _row(r, carry):
            for c in range(D // 16):
                sl = pl.ds(c * 16, 16)
                b0_v[r, sl] = b0_v[r, sl] + b1_v[r, sl]
            return carry

        lax.fori_loop(0, TPW, _row, 0)
        pltpu.sync_copy(b0_v, out_hbm.at[pl.ds(base, TPW)])

    return k


def _dispatch(xf, pad_pos, gflat):
    tok1 = jnp.arange(NP, dtype=jnp.int32) // TOPK + 1
    zi = jnp.zeros((P,), jnp.int32)
    zf = jnp.zeros((P,), jnp.float32)
    pp3 = pad_pos.reshape(NS, SPW // RC, RC)
    return _sc_dispatch_kernel()(xf, pp3, tok1, gflat, zi, zf)


def _combine_rows(y, d0, d1):
    return _sc_combine_kernel()(y, d0, d1)


def kernel(x, router_w, w1, w2):
    B, S, _ = x.shape
    xf = x.reshape(T, D)

    logits, eidx, gates = pl.pallas_call(
        _router_kernel,
        out_shape=(
            jax.ShapeDtypeStruct((T, E), jnp.float32),
            jax.ShapeDtypeStruct((T, TOPK), jnp.int32),
            jax.ShapeDtypeStruct((T, TOPK), jnp.float32),
        ),
    )(xf, router_w.T)

    eflat = eidx.reshape(-1)
    gflat = gates.reshape(-1)
    onehot = (eflat[:, None] == jnp.arange(E, dtype=jnp.int32)).astype(jnp.int32)
    csum = jnp.cumsum(onehot, axis=0)
    rank = jnp.sum(onehot * csum, axis=1) - 1
    counts = csum[-1]
    pcounts = ((counts + BM - 1) // BM) * BM
    cum_p = jnp.cumsum(pcounts)
    pstart = cum_p - pcounts
    pad_pos = jnp.sum(onehot * pstart[None, :], axis=1) + rank
    dst = pad_pos.reshape(T, TOPK)
    be_raw = jnp.searchsorted(cum_p, jnp.arange(NB, dtype=jnp.int32) * BM,
                              side="right").astype(jnp.int32)
    last_e = jnp.searchsorted(cum_p, cum_p[-1] - 1,
                              side="right").astype(jnp.int32)
    block_expert = jnp.where(be_raw >= E, last_e, be_raw)
    nvb = (cum_p[-1:] // BM).astype(jnp.int32)

    x_sorted, gate_sorted = _dispatch(xf, pad_pos, gflat)

    grid_spec = pltpu.PrefetchScalarGridSpec(
        num_scalar_prefetch=2,
        grid=(NB,),
        in_specs=[
            pl.BlockSpec((BM, D), lambda b, be, nv: (b, 0)),
            pl.BlockSpec((D, F), lambda b, be, nv: (0, be[b])),
            pl.BlockSpec((F, D), lambda b, be, nv: (be[b], 0)),
            pl.BlockSpec((1, 1, BM), lambda b, be, nv: (b, 0, 0)),
        ],
        out_specs=pl.BlockSpec((BM, D), lambda b, be, nv: (b, 0)),
    )
    y = pl.pallas_call(
        _ffn_kernel,
        grid_spec=grid_spec,
        out_shape=jax.ShapeDtypeStruct((P, D), jnp.float32),
    )(block_expert, nvb, x_sorted, w1, w2, gate_sorted.reshape(NB, 1, BM))

    out = _combine_rows(y, dst[:, 0], dst[:, 1])
    return out.reshape(B, S, D), logits
